# Initial kernel scaffold; baseline (speedup 1.0000x reference)
import functools

import jax
import jax.numpy as jnp
from jax import lax
from jax.experimental import pallas as pl
from jax.experimental.pallas import tpu as pltpu
from jax.experimental.pallas import tpu_sc as plsc

NN = 10000
EE = 320000
EMB = 128
NUM_GRAPHS = 64
LANES = 16
NCORES = 2
NSUB = 16
NW = NCORES * NSUB
CHUNK = 128
SLAB_ROWS = 10016
ROWS_PER_TILE = SLAB_ROWS // NW
DEAD_ROW = 10008

E_CONV = NW * CHUNK * 81
CONV_CHUNKS = 81
E_DEG = NW * CHUNK * 79
DEG_CHUNKS = 79

MBLK = 1000


def _mm1_body(a_ref, w_ref, b_ref, o_ref):
    o_ref[...] = (
        jnp.dot(a_ref[...], w_ref[...], preferred_element_type=jnp.float32)
        + b_ref[...]
    )


def _mm1(a, w, b):
    m, kdim = a.shape
    n = w.shape[1]
    return pl.pallas_call(
        _mm1_body,
        grid=(m // MBLK,),
        in_specs=[
            pl.BlockSpec((MBLK, kdim), lambda i: (i, 0)),
            pl.BlockSpec((kdim, n), lambda i: (0, 0)),
            pl.BlockSpec((1, n), lambda i: (0, 0)),
        ],
        out_specs=pl.BlockSpec((MBLK, n), lambda i: (i, 0)),
        out_shape=jax.ShapeDtypeStruct((m, n), jnp.float32),
    )(a, w, b.reshape(1, n))


def _mm2_body(p0_ref, p1_ref, w_ref, b_ref, o_ref):
    a = jnp.maximum(p0_ref[...] + p1_ref[...], 0.0)
    o_ref[...] = (
        jnp.dot(a, w_ref[...], preferred_element_type=jnp.float32) + b_ref[...]
    )


def _mm2(p0, p1, w, b):
    n = w.shape[1]
    return pl.pallas_call(
        _mm2_body,
        grid=(NN // MBLK,),
        in_specs=[
            pl.BlockSpec((MBLK, EMB), lambda i: (i, 0)),
            pl.BlockSpec((MBLK, EMB), lambda i: (i, 0)),
            pl.BlockSpec((EMB, n), lambda i: (0, 0)),
            pl.BlockSpec((1, n), lambda i: (0, 0)),
        ],
        out_specs=pl.BlockSpec((MBLK, n), lambda i: (i, 0)),
        out_shape=jax.ShapeDtypeStruct((NN, n), jnp.float32),
    )(p0, p1, w, b.reshape(1, n))


def _pool_body(p00, p01, p10, p11, b3, wp, bp, o_ref, s_ref, c_ref):
    i = pl.program_id(0)

    @pl.when(i == 0)
    def _init():
        s_ref[...] = jnp.zeros_like(s_ref)
        c_ref[...] = jnp.zeros_like(c_ref)

    h = p00[...] + p01[...] + p10[...] + p11[...]
    bt = b3[0, 0, :]
    onehot = (
        bt[None, :] == lax.broadcasted_iota(jnp.int32, (NUM_GRAPHS, MBLK), 0)
    ).astype(jnp.float32)
    s_ref[...] += jnp.dot(onehot, h, preferred_element_type=jnp.float32)
    c_ref[...] += jnp.broadcast_to(
        jnp.sum(onehot, axis=1, keepdims=True), (NUM_GRAPHS, EMB)
    )

    @pl.when(i == pl.num_programs(0) - 1)
    def _fin():
        r = s_ref[...] / jnp.maximum(c_ref[...], 1.0)
        o_ref[...] = (
            jnp.dot(r, wp[...], preferred_element_type=jnp.float32) + bp[...]
        )


def _pool(p00, p01, p10, p11, batch3, wp, bp):
    ntasks = wp.shape[1]
    return pl.pallas_call(
        _pool_body,
        grid=(NN // MBLK,),
        in_specs=[
            pl.BlockSpec((MBLK, EMB), lambda i: (i, 0)),
            pl.BlockSpec((MBLK, EMB), lambda i: (i, 0)),
            pl.BlockSpec((MBLK, EMB), lambda i: (i, 0)),
            pl.BlockSpec((MBLK, EMB), lambda i: (i, 0)),
            pl.BlockSpec((1, 1, MBLK), lambda i: (i, 0, 0)),
            pl.BlockSpec((EMB, ntasks), lambda i: (0, 0)),
            pl.BlockSpec((1, ntasks), lambda i: (0, 0)),
        ],
        out_specs=pl.BlockSpec((NUM_GRAPHS, ntasks), lambda i: (0, 0)),
        out_shape=jax.ShapeDtypeStruct((NUM_GRAPHS, ntasks), jnp.float32),
        scratch_shapes=[
            pltpu.VMEM((NUM_GRAPHS, EMB), jnp.float32),
            pltpu.VMEM((NUM_GRAPHS, EMB), jnp.float32),
        ],
    )(p00, p01, p10, p11, batch3, wp, bp.reshape(1, ntasks))


def _sc_mesh():
    return plsc.VectorSubcoreMesh(
        core_axis_name="c", subcore_axis_name="s",
        num_cores=NCORES, num_subcores=NSUB,
    )


def _deg_body(dst_hbm, out_hbm, slab, idx_v, ones_v, zbuf, sem):
    cid = lax.axis_index("c")
    sid = lax.axis_index("s")
    wid = sid * NCORES + cid

    def _zrow(r, carry):
        zbuf[r, :] = jnp.zeros((LANES,), jnp.float32)
        return carry

    lax.fori_loop(0, ROWS_PER_TILE, _zrow, 0)

    def _orow(r, carry):
        ones_v[r, :] = jnp.ones((LANES,), jnp.float32)
        return carry

    lax.fori_loop(0, CHUNK, _orow, 0)

    pltpu.sync_copy(zbuf, slab.at[pl.ds(sid * ROWS_PER_TILE, ROWS_PER_TILE), :])
    plsc.subcore_barrier()

    def _chunk(t, carry):
        base = wid * (DEG_CHUNKS * CHUNK) + t * CHUNK
        pltpu.sync_copy(dst_hbm.at[pl.ds(base, CHUNK)], idx_v)
        pltpu.sync_copy(ones_v, slab.at[idx_v], add=True)
        return carry

    lax.fori_loop(0, DEG_CHUNKS, _chunk, 0)
    plsc.subcore_barrier()
    pltpu.sync_copy(
        slab.at[pl.ds(sid * ROWS_PER_TILE, ROWS_PER_TILE), :],
        out_hbm.at[cid].at[pl.ds(sid * ROWS_PER_TILE, ROWS_PER_TILE), :],
    )


def _deg(dst_pad):
    k = pl.kernel(
        _deg_body,
        out_type=jax.ShapeDtypeStruct((NCORES, SLAB_ROWS, LANES), jnp.float32),
        mesh=_sc_mesh(),
        scratch_types=[
            pltpu.VMEM_SHARED((SLAB_ROWS, LANES), jnp.float32),
            pltpu.VMEM((CHUNK,), jnp.int32),
            pltpu.VMEM((CHUNK, LANES), jnp.float32),
            pltpu.VMEM((ROWS_PER_TILE, LANES), jnp.float32),
            pltpu.SemaphoreType.DMA,
        ],
    )
    return k(dst_pad)


def _conv_body(src_hbm, dst_hbm, attr_hbm, rs_hbm, hw_hbm, we_hbm, out_hbm,
               slab, rs_v, we_v, idx_s, idx_d, attr_v, norm_v, rows_v, msg_v,
               zbuf, sem):
    cid = lax.axis_index("c")
    sid = lax.axis_index("s")
    wid = sid * NCORES + cid

    pltpu.sync_copy(rs_hbm, rs_v)
    pltpu.sync_copy(we_hbm, we_v)

    def _zrow(r, carry):
        for q in range(EMB // LANES):
            zbuf[r, pl.ds(q * LANES, LANES)] = jnp.zeros((LANES,), jnp.float32)
        return carry

    lax.fori_loop(0, ROWS_PER_TILE, _zrow, 0)
    pltpu.sync_copy(zbuf, slab.at[pl.ds(sid * ROWS_PER_TILE, ROWS_PER_TILE), :])
    plsc.subcore_barrier()

    def _chunk(t, carry):
        base = wid * (CONV_CHUNKS * CHUNK) + t * CHUNK
        pltpu.sync_copy(src_hbm.at[pl.ds(base, CHUNK)], idx_s)
        pltpu.sync_copy(dst_hbm.at[pl.ds(base, CHUNK)], idx_d)
        pltpu.sync_copy(attr_hbm.at[pl.ds(base, CHUNK), :], attr_v)
        cp = pltpu.async_copy(hw_hbm.at[idx_s], rows_v, sem)

        for g in range(CHUNK // LANES):
            sl = pl.ds(g * LANES, LANES)
            sv = idx_s[sl]
            dv = idx_d[sl]
            norm_v[sl] = plsc.load_gather(rs_v, [sv]) * plsc.load_gather(rs_v, [dv])

        cp.wait()

        def _edge(j, ecarry):
            a0 = attr_v[j, 0]
            a1 = attr_v[j, 1]
            a2 = attr_v[j, 2]
            a3 = attr_v[j, 3]
            nj = norm_v[j]
            for q in range(EMB // LANES):
                sl = pl.ds(q * LANES, LANES)
                e = (a0 * we_v[0, sl] + a1 * we_v[1, sl]
                     + a2 * we_v[2, sl] + a3 * we_v[3, sl])
                msg_v[j, sl] = jnp.maximum(rows_v[j, sl] + e, 0.0) * nj
            return ecarry

        lax.fori_loop(0, CHUNK, _edge, 0)
        pltpu.sync_copy(msg_v, slab.at[idx_d], add=True)
        return carry

    lax.fori_loop(0, CONV_CHUNKS, _chunk, 0)
    plsc.subcore_barrier()
    pltpu.sync_copy(
        slab.at[pl.ds(sid * ROWS_PER_TILE, ROWS_PER_TILE), :],
        out_hbm.at[cid].at[pl.ds(sid * ROWS_PER_TILE, ROWS_PER_TILE), :],
    )


def _conv(src_pad, dst_pad, attr_pad, rs, hw, we_l):
    k = pl.kernel(
        _conv_body,
        out_type=jax.ShapeDtypeStruct((NCORES, SLAB_ROWS, EMB), jnp.float32),
        mesh=_sc_mesh(),
        scratch_types=[
            pltpu.VMEM_SHARED((SLAB_ROWS, EMB), jnp.float32),
            pltpu.VMEM((SLAB_ROWS,), jnp.float32),
            pltpu.VMEM((4, EMB), jnp.float32),
            pltpu.VMEM((CHUNK,), jnp.int32),
            pltpu.VMEM((CHUNK,), jnp.int32),
            pltpu.VMEM((CHUNK, 4), jnp.float32),
            pltpu.VMEM((CHUNK,), jnp.float32),
            pltpu.VMEM((CHUNK, EMB), jnp.float32),
            pltpu.VMEM((CHUNK, EMB), jnp.float32),
            pltpu.VMEM((ROWS_PER_TILE, EMB), jnp.float32),
            pltpu.SemaphoreType.DMA,
        ],
    )
    return k(src_pad, dst_pad, attr_pad, rs, hw, we_l)


def kernel(x, edge_index, edge_attr, batch, W_enc, b_enc, Wg, bg, We,
           W_pred, b_pred):
    src = edge_index[0]
    dst = edge_index[1]
    iota = jnp.arange(NN, dtype=jnp.int32)

    npad = E_CONV - EE - NN
    src_pad = jnp.concatenate([src, iota, jnp.zeros((npad,), jnp.int32)])
    dst_pad = jnp.concatenate(
        [dst, iota, jnp.full((npad,), DEAD_ROW, jnp.int32)]
    )
    attr_pad = jnp.concatenate(
        [edge_attr, jnp.zeros((NN + npad, 4), jnp.float32)]
    )
    dst_deg = jnp.concatenate(
        [dst, jnp.full((E_DEG - EE,), DEAD_ROW, jnp.int32)]
    )

    degp = _deg(dst_deg)
    deg = degp[0, :NN, 0] + degp[1, :NN, 0] + 1.0
    rs = jnp.concatenate(
        [lax.rsqrt(deg), jnp.zeros((SLAB_ROWS - NN,), jnp.float32)]
    )

    feat = _mm1(x, W_enc, b_enc)

    parts = []
    for i in range(2):
        hw0 = _mm1(feat, Wg[i, 0], bg[i, 0])
        p = _conv(src_pad, dst_pad, attr_pad, rs, hw0, We[i, 0])
        hw1 = _mm2(p[0], p[1], Wg[i, 1], bg[i, 1])
        p2 = _conv(src_pad, dst_pad, attr_pad, rs, hw1, We[i, 1])
        parts.append(p2)

    batch3 = batch.reshape(NN // MBLK, 1, MBLK)
    return _pool(parts[0][0], parts[0][1], parts[1][0], parts[1][1],
                 batch3, W_pred, b_pred)


# speedup vs baseline: 1.5882x; 1.5882x over previous
import functools

import jax
import jax.numpy as jnp
from jax import lax
from jax.experimental import pallas as pl
from jax.experimental.pallas import tpu as pltpu
from jax.experimental.pallas import tpu_sc as plsc

NN = 10000
EE = 320000
EMB = 128
NUM_GRAPHS = 64
LANES = 16
NCORES = 2
NSUB = 16
NW = NCORES * NSUB
CHUNK = 128
SLAB_ROWS = 10240
ROWS_PER_TILE = SLAB_ROWS // NW
DEAD_ROW = 10008

E_CONV = NW * CHUNK * 81
CONV_CHUNKS = 81
E_DEG = NW * CHUNK * 79
DEG_CHUNKS = 79

MBLK = 1000


def _mm1_body(a_ref, w_ref, b_ref, o_ref):
    o_ref[...] = (
        jnp.dot(a_ref[...], w_ref[...], preferred_element_type=jnp.float32)
        + b_ref[...]
    )


def _mm1(a, w, b):
    m, kdim = a.shape
    n = w.shape[1]
    return pl.pallas_call(
        _mm1_body,
        grid=(m // MBLK,),
        in_specs=[
            pl.BlockSpec((MBLK, kdim), lambda i: (i, 0)),
            pl.BlockSpec((kdim, n), lambda i: (0, 0)),
            pl.BlockSpec((1, n), lambda i: (0, 0)),
        ],
        out_specs=pl.BlockSpec((MBLK, n), lambda i: (i, 0)),
        out_shape=jax.ShapeDtypeStruct((m, n), jnp.float32),
    )(a, w, b.reshape(1, n))


def _mm2_body(p0_ref, p1_ref, w_ref, b_ref, o_ref):
    a = jnp.maximum(p0_ref[...] + p1_ref[...], 0.0)
    o_ref[...] = (
        jnp.dot(a, w_ref[...], preferred_element_type=jnp.float32) + b_ref[...]
    )


def _mm2(p0, p1, w, b):
    n = w.shape[1]
    return pl.pallas_call(
        _mm2_body,
        grid=(NN // MBLK,),
        in_specs=[
            pl.BlockSpec((MBLK, EMB), lambda i: (i, 0)),
            pl.BlockSpec((MBLK, EMB), lambda i: (i, 0)),
            pl.BlockSpec((EMB, n), lambda i: (0, 0)),
            pl.BlockSpec((1, n), lambda i: (0, 0)),
        ],
        out_specs=pl.BlockSpec((MBLK, n), lambda i: (i, 0)),
        out_shape=jax.ShapeDtypeStruct((NN, n), jnp.float32),
    )(p0, p1, w, b.reshape(1, n))


def _rs_body(d_ref, o_ref):
    d = d_ref[0] + d_ref[1] + 1.0
    n = (
        lax.broadcasted_iota(jnp.int32, (SLAB_ROWS // 128, 128), 0) * 128
        + lax.broadcasted_iota(jnp.int32, (SLAB_ROWS // 128, 128), 1)
    )
    o_ref[...] = jnp.where(n < NN, lax.rsqrt(d), 0.0)


def _rs(dcol):
    rows = SLAB_ROWS // 128
    return pl.pallas_call(
        _rs_body,
        in_specs=[pl.BlockSpec((2, rows, 128), lambda: (0, 0, 0))],
        out_specs=pl.BlockSpec((rows, 128), lambda: (0, 0)),
        out_shape=jax.ShapeDtypeStruct((rows, 128), jnp.float32),
    )(dcol)


def _pool_body(p00, p01, p10, p11, b3, wp, bp, o_ref, s_ref, c_ref):
    i = pl.program_id(0)

    @pl.when(i == 0)
    def _init():
        s_ref[...] = jnp.zeros_like(s_ref)
        c_ref[...] = jnp.zeros_like(c_ref)

    h = p00[...] + p01[...] + p10[...] + p11[...]
    bt = b3[0, 0, :]
    onehot = (
        bt[None, :] == lax.broadcasted_iota(jnp.int32, (NUM_GRAPHS, MBLK), 0)
    ).astype(jnp.float32)
    s_ref[...] += jnp.dot(onehot, h, preferred_element_type=jnp.float32)
    c_ref[...] += jnp.broadcast_to(
        jnp.sum(onehot, axis=1, keepdims=True), (NUM_GRAPHS, EMB)
    )

    @pl.when(i == pl.num_programs(0) - 1)
    def _fin():
        r = s_ref[...] / jnp.maximum(c_ref[...], 1.0)
        o_ref[...] = (
            jnp.dot(r, wp[...], preferred_element_type=jnp.float32) + bp[...]
        )


def _pool(p00, p01, p10, p11, batch3, wp, bp):
    ntasks = wp.shape[1]
    return pl.pallas_call(
        _pool_body,
        grid=(NN // MBLK,),
        in_specs=[
            pl.BlockSpec((MBLK, EMB), lambda i: (i, 0)),
            pl.BlockSpec((MBLK, EMB), lambda i: (i, 0)),
            pl.BlockSpec((MBLK, EMB), lambda i: (i, 0)),
            pl.BlockSpec((MBLK, EMB), lambda i: (i, 0)),
            pl.BlockSpec((1, 1, MBLK), lambda i: (i, 0, 0)),
            pl.BlockSpec((EMB, ntasks), lambda i: (0, 0)),
            pl.BlockSpec((1, ntasks), lambda i: (0, 0)),
        ],
        out_specs=pl.BlockSpec((NUM_GRAPHS, ntasks), lambda i: (0, 0)),
        out_shape=jax.ShapeDtypeStruct((NUM_GRAPHS, ntasks), jnp.float32),
        scratch_shapes=[
            pltpu.VMEM((NUM_GRAPHS, EMB), jnp.float32),
            pltpu.VMEM((NUM_GRAPHS, EMB), jnp.float32),
        ],
    )(p00, p01, p10, p11, batch3, wp, bp.reshape(1, ntasks))


def _sc_mesh():
    return plsc.VectorSubcoreMesh(
        core_axis_name="c", subcore_axis_name="s",
        num_cores=NCORES, num_subcores=NSUB,
    )


def _deg_body(dst_hbm, out_hbm, slab, idx_v, ones_v, zbuf, sem):
    cid = lax.axis_index("c")
    sid = lax.axis_index("s")
    wid = sid * NCORES + cid

    def _zrow(r, carry):
        zbuf[r, :] = jnp.zeros((LANES,), jnp.float32)
        return carry

    lax.fori_loop(0, ROWS_PER_TILE, _zrow, 0)

    def _orow(r, carry):
        ones_v[r, :] = jnp.ones((LANES,), jnp.float32)
        return carry

    lax.fori_loop(0, CHUNK, _orow, 0)

    pltpu.sync_copy(zbuf, slab.at[pl.ds(sid * ROWS_PER_TILE, ROWS_PER_TILE), :])
    plsc.subcore_barrier()

    def _chunk(t, carry):
        base = wid * (DEG_CHUNKS * CHUNK) + t * CHUNK
        pltpu.sync_copy(dst_hbm.at[pl.ds(base, CHUNK)], idx_v)
        pltpu.sync_copy(ones_v, slab.at[idx_v], add=True)
        return carry

    lax.fori_loop(0, DEG_CHUNKS, _chunk, 0)
    plsc.subcore_barrier()
    pltpu.sync_copy(
        slab.at[pl.ds(sid * ROWS_PER_TILE, ROWS_PER_TILE), :],
        out_hbm.at[cid].at[pl.ds(sid * ROWS_PER_TILE, ROWS_PER_TILE), :],
    )


def _deg(dst_pad):
    k = pl.kernel(
        _deg_body,
        out_type=pltpu.HBM((NCORES, SLAB_ROWS, LANES), jnp.float32),
        mesh=_sc_mesh(),
        scratch_types=[
            pltpu.VMEM_SHARED((SLAB_ROWS, LANES), jnp.float32),
            pltpu.VMEM((CHUNK,), jnp.int32),
            pltpu.VMEM((CHUNK, LANES), jnp.float32),
            pltpu.VMEM((ROWS_PER_TILE, LANES), jnp.float32),
            pltpu.SemaphoreType.DMA,
        ],
    )
    return k(dst_pad)


def _conv_body(src_hbm, dst_hbm, attr_hbm, rs_hbm, hw_hbm, we_hbm, out_hbm,
               slab, rs_v, we_v, idx_s, idx_d, attr_v, norm_v, rows_v, msg_v,
               sem):
    cid = lax.axis_index("c")
    sid = lax.axis_index("s")
    wid = sid * NCORES + cid

    pltpu.sync_copy(rs_hbm, rs_v)
    pltpu.sync_copy(we_hbm, we_v)

    def _zrow(r, carry):
        for q in range(EMB // LANES):
            msg_v[r, pl.ds(q * LANES, LANES)] = jnp.zeros((LANES,), jnp.float32)
        return carry

    lax.fori_loop(0, CHUNK, _zrow, 0)
    base_row = sid * ROWS_PER_TILE
    for b in range(ROWS_PER_TILE // CHUNK):
        pltpu.sync_copy(msg_v, slab.at[pl.ds(base_row + b * CHUNK, CHUNK), :])
    rem = ROWS_PER_TILE % CHUNK
    if rem:
        pltpu.sync_copy(
            msg_v.at[pl.ds(0, rem), :],
            slab.at[pl.ds(base_row + (ROWS_PER_TILE // CHUNK) * CHUNK, rem), :],
        )
    plsc.subcore_barrier()

    def _chunk(t, carry):
        base = wid * (CONV_CHUNKS * CHUNK) + t * CHUNK
        pltpu.sync_copy(src_hbm.at[pl.ds(base, CHUNK)], idx_s)
        pltpu.sync_copy(dst_hbm.at[pl.ds(base, CHUNK)], idx_d)
        pltpu.sync_copy(attr_hbm.at[pl.ds(base * 4, CHUNK * 4)],
                        attr_v.at[pl.ds(0, CHUNK * 4)])
        cp = pltpu.async_copy(hw_hbm.at[idx_s], rows_v, sem)

        for g in range(CHUNK // LANES):
            sl = pl.ds(g * LANES, LANES)
            sv = idx_s[sl]
            dv = idx_d[sl]
            norm_v[sl] = plsc.load_gather(rs_v, [sv]) * plsc.load_gather(rs_v, [dv])

        cp.wait()

        def _edge(j, ecarry):
            av = attr_v[pl.ds(j * 4, LANES)]
            a0 = av[0]
            a1 = av[1]
            a2 = av[2]
            a3 = av[3]
            nj = norm_v[pl.ds(j, LANES)][0]
            for q in range(EMB // LANES):
                sl = pl.ds(q * LANES, LANES)
                e = (a0 * we_v[0, sl] + a1 * we_v[1, sl]
                     + a2 * we_v[2, sl] + a3 * we_v[3, sl])
                msg_v[j, sl] = jnp.maximum(rows_v[j, sl] + e, 0.0) * nj
            return ecarry

        lax.fori_loop(0, CHUNK, _edge, 0)
        pltpu.sync_copy(msg_v, slab.at[idx_d], add=True)
        return carry

    lax.fori_loop(0, CONV_CHUNKS, _chunk, 0)
    plsc.subcore_barrier()
    pltpu.sync_copy(
        slab.at[pl.ds(sid * ROWS_PER_TILE, ROWS_PER_TILE), :],
        out_hbm.at[cid].at[pl.ds(sid * ROWS_PER_TILE, ROWS_PER_TILE), :],
    )


def _conv(src_pad, dst_pad, attr_pad, rs, hw, we_l):
    k = pl.kernel(
        _conv_body,
        out_type=pltpu.HBM((NCORES, SLAB_ROWS, EMB), jnp.float32),
        mesh=_sc_mesh(),
        compiler_params=pltpu.CompilerParams(needs_layout_passes=False),
        scratch_types=[
            pltpu.VMEM_SHARED((SLAB_ROWS, EMB), jnp.float32),
            pltpu.VMEM((SLAB_ROWS,), jnp.float32),
            pltpu.VMEM((4, EMB), jnp.float32),
            pltpu.VMEM((CHUNK,), jnp.int32),
            pltpu.VMEM((CHUNK,), jnp.int32),
            pltpu.VMEM((CHUNK * 4 + LANES,), jnp.float32),
            pltpu.VMEM((CHUNK + LANES,), jnp.float32),
            pltpu.VMEM((CHUNK, EMB), jnp.float32),
            pltpu.VMEM((CHUNK, EMB), jnp.float32),
            pltpu.SemaphoreType.DMA,
        ],
    )
    return k(src_pad, dst_pad, attr_pad, rs, hw, we_l)


def kernel(x, edge_index, edge_attr, batch, W_enc, b_enc, Wg, bg, We,
           W_pred, b_pred):
    src = edge_index[0]
    dst = edge_index[1]
    iota = jnp.arange(NN, dtype=jnp.int32)

    npad = E_CONV - EE - NN
    src_pad = jnp.concatenate([src, iota, jnp.zeros((npad,), jnp.int32)])
    dst_pad = jnp.concatenate(
        [dst, iota, jnp.full((npad,), DEAD_ROW, jnp.int32)]
    )
    attr_pad = jnp.concatenate(
        [edge_attr, jnp.zeros((NN + npad, 4), jnp.float32)]
    ).reshape(-1)
    dst_deg = jnp.concatenate(
        [dst, jnp.full((E_DEG - EE,), DEAD_ROW, jnp.int32)]
    )

    degp = _deg(dst_deg)
    dcol = degp[:, :, 0].reshape(2, SLAB_ROWS // 128, 128)
    rs = _rs(dcol).reshape(SLAB_ROWS)

    feat = _mm1(x, W_enc, b_enc)

    parts = []
    for i in range(2):
        hw0 = _mm1(feat, Wg[i, 0], bg[i, 0])
        p = _conv(src_pad, dst_pad, attr_pad, rs, hw0, We[i, 0])
        hw1 = _mm2(p[0], p[1], Wg[i, 1], bg[i, 1])
        p2 = _conv(src_pad, dst_pad, attr_pad, rs, hw1, We[i, 1])
        parts.append(p2)

    batch3 = batch.reshape(NN // MBLK, 1, MBLK)
    return _pool(parts[0][0], parts[0][1], parts[1][0], parts[1][1],
                 batch3, W_pred, b_pred)


# speedup vs baseline: 3.1533x; 1.9854x over previous
import functools

import jax
import jax.numpy as jnp
from jax import lax
from jax.experimental import pallas as pl
from jax.experimental.pallas import tpu as pltpu
from jax.experimental.pallas import tpu_sc as plsc

NN = 10000
EE = 320000
EMB = 128
NUM_GRAPHS = 64
LANES = 16
NCORES = 2
NSUB = 16
NW = NCORES * NSUB
CHUNK = 128
SLAB_ROWS = 10240
ROWS_PER_TILE = SLAB_ROWS // NW
DEAD_ROW = 10008

E_CONV = NW * CHUNK * 81
CONV_CHUNKS = 81
E_DEG = NW * CHUNK * 79
DEG_CHUNKS = 79

MBLK = 1000


def _mm1_body(a_ref, w_ref, b_ref, o_ref):
    o_ref[...] = (
        jnp.dot(a_ref[...], w_ref[...], preferred_element_type=jnp.float32)
        + b_ref[...]
    )


def _mm1(a, w, b):
    m, kdim = a.shape
    n = w.shape[1]
    return pl.pallas_call(
        _mm1_body,
        grid=(m // MBLK,),
        in_specs=[
            pl.BlockSpec((MBLK, kdim), lambda i: (i, 0)),
            pl.BlockSpec((kdim, n), lambda i: (0, 0)),
            pl.BlockSpec((1, n), lambda i: (0, 0)),
        ],
        out_specs=pl.BlockSpec((MBLK, n), lambda i: (i, 0)),
        out_shape=jax.ShapeDtypeStruct((m, n), jnp.float32),
    )(a, w, b.reshape(1, n))


def _mm2_body(p0_ref, p1_ref, w_ref, b_ref, o_ref):
    a = jnp.maximum(p0_ref[...] + p1_ref[...], 0.0)
    o_ref[...] = (
        jnp.dot(a, w_ref[...], preferred_element_type=jnp.float32) + b_ref[...]
    )


def _mm2(p0, p1, w, b):
    n = w.shape[1]
    return pl.pallas_call(
        _mm2_body,
        grid=(NN // MBLK,),
        in_specs=[
            pl.BlockSpec((MBLK, EMB), lambda i: (i, 0)),
            pl.BlockSpec((MBLK, EMB), lambda i: (i, 0)),
            pl.BlockSpec((EMB, n), lambda i: (0, 0)),
            pl.BlockSpec((1, n), lambda i: (0, 0)),
        ],
        out_specs=pl.BlockSpec((MBLK, n), lambda i: (i, 0)),
        out_shape=jax.ShapeDtypeStruct((NN, n), jnp.float32),
    )(p0, p1, w, b.reshape(1, n))


def _rs_body(d_ref, o_ref):
    d = d_ref[0] + d_ref[1] + 1.0
    n = (
        lax.broadcasted_iota(jnp.int32, (SLAB_ROWS // 128, 128), 0) * 128
        + lax.broadcasted_iota(jnp.int32, (SLAB_ROWS // 128, 128), 1)
    )
    o_ref[...] = jnp.where(n < NN, lax.rsqrt(d), 0.0)


def _rs(dcol):
    rows = SLAB_ROWS // 128
    return pl.pallas_call(
        _rs_body,
        in_specs=[pl.BlockSpec((2, rows, 128), lambda: (0, 0, 0))],
        out_specs=pl.BlockSpec((rows, 128), lambda: (0, 0)),
        out_shape=jax.ShapeDtypeStruct((rows, 128), jnp.float32),
    )(dcol)


def _pool_body(p00, p01, p10, p11, b3, wp, bp, o_ref, s_ref, c_ref):
    i = pl.program_id(0)

    @pl.when(i == 0)
    def _init():
        s_ref[...] = jnp.zeros_like(s_ref)
        c_ref[...] = jnp.zeros_like(c_ref)

    h = p00[...] + p01[...] + p10[...] + p11[...]
    bt = b3[0, 0, :]
    onehot = (
        bt[None, :] == lax.broadcasted_iota(jnp.int32, (NUM_GRAPHS, MBLK), 0)
    ).astype(jnp.float32)
    s_ref[...] += jnp.dot(onehot, h, preferred_element_type=jnp.float32)
    c_ref[...] += jnp.broadcast_to(
        jnp.sum(onehot, axis=1, keepdims=True), (NUM_GRAPHS, EMB)
    )

    @pl.when(i == pl.num_programs(0) - 1)
    def _fin():
        r = s_ref[...] / jnp.maximum(c_ref[...], 1.0)
        o_ref[...] = (
            jnp.dot(r, wp[...], preferred_element_type=jnp.float32) + bp[...]
        )


def _pool(p00, p01, p10, p11, batch3, wp, bp):
    ntasks = wp.shape[1]
    return pl.pallas_call(
        _pool_body,
        grid=(NN // MBLK,),
        in_specs=[
            pl.BlockSpec((MBLK, EMB), lambda i: (i, 0)),
            pl.BlockSpec((MBLK, EMB), lambda i: (i, 0)),
            pl.BlockSpec((MBLK, EMB), lambda i: (i, 0)),
            pl.BlockSpec((MBLK, EMB), lambda i: (i, 0)),
            pl.BlockSpec((1, 1, MBLK), lambda i: (i, 0, 0)),
            pl.BlockSpec((EMB, ntasks), lambda i: (0, 0)),
            pl.BlockSpec((1, ntasks), lambda i: (0, 0)),
        ],
        out_specs=pl.BlockSpec((NUM_GRAPHS, ntasks), lambda i: (0, 0)),
        out_shape=jax.ShapeDtypeStruct((NUM_GRAPHS, ntasks), jnp.float32),
        scratch_shapes=[
            pltpu.VMEM((NUM_GRAPHS, EMB), jnp.float32),
            pltpu.VMEM((NUM_GRAPHS, EMB), jnp.float32),
        ],
    )(p00, p01, p10, p11, batch3, wp, bp.reshape(1, ntasks))


def _sc_mesh():
    return plsc.VectorSubcoreMesh(
        core_axis_name="c", subcore_axis_name="s",
        num_cores=NCORES, num_subcores=NSUB,
    )


def _deg_body(dst_hbm, out_hbm, slab, idx_v, ones_v, zbuf, sem):
    cid = lax.axis_index("c")
    sid = lax.axis_index("s")
    wid = sid * NCORES + cid

    def _zrow(r, carry):
        zbuf[r, :] = jnp.zeros((LANES,), jnp.float32)
        return carry

    lax.fori_loop(0, ROWS_PER_TILE, _zrow, 0)

    def _orow(r, carry):
        ones_v[r, :] = jnp.ones((LANES,), jnp.float32)
        return carry

    lax.fori_loop(0, CHUNK, _orow, 0)

    pltpu.sync_copy(zbuf, slab.at[pl.ds(sid * ROWS_PER_TILE, ROWS_PER_TILE), :])
    plsc.subcore_barrier()

    def _chunk(t, carry):
        base = wid * (DEG_CHUNKS * CHUNK) + t * CHUNK
        pltpu.sync_copy(dst_hbm.at[pl.ds(base, CHUNK)], idx_v)
        pltpu.sync_copy(ones_v, slab.at[idx_v], add=True)
        return carry

    lax.fori_loop(0, DEG_CHUNKS, _chunk, 0)
    plsc.subcore_barrier()
    pltpu.sync_copy(
        slab.at[pl.ds(sid * ROWS_PER_TILE, ROWS_PER_TILE), :],
        out_hbm.at[cid].at[pl.ds(sid * ROWS_PER_TILE, ROWS_PER_TILE), :],
    )


def _deg(dst_pad):
    k = pl.kernel(
        _deg_body,
        out_type=pltpu.HBM((NCORES, SLAB_ROWS, LANES), jnp.float32),
        mesh=_sc_mesh(),
        scratch_types=[
            pltpu.VMEM_SHARED((SLAB_ROWS, LANES), jnp.float32),
            pltpu.VMEM((CHUNK,), jnp.int32),
            pltpu.VMEM((CHUNK, LANES), jnp.float32),
            pltpu.VMEM((ROWS_PER_TILE, LANES), jnp.float32),
            pltpu.SemaphoreType.DMA,
        ],
    )
    return k(dst_pad)


def _conv_body(src_hbm, dst_hbm, attr_hbm, rs_hbm, hw_hbm, we_hbm, out_hbm,
               slab, rs_v, we_v, idx_s, idx_d, attr_v, norm_v, rows_v, msg_v,
               sem):
    cid = lax.axis_index("c")
    sid = lax.axis_index("s")
    wid = sid * NCORES + cid

    pltpu.sync_copy(rs_hbm, rs_v)
    pltpu.sync_copy(we_hbm, we_v)

    def _zrow(r, carry):
        for q in range(EMB // LANES):
            msg_v[r, pl.ds(q * LANES, LANES)] = jnp.zeros((LANES,), jnp.float32)
        return carry

    lax.fori_loop(0, CHUNK, _zrow, 0)
    base_row = sid * ROWS_PER_TILE
    for b in range(ROWS_PER_TILE // CHUNK):
        pltpu.sync_copy(msg_v, slab.at[pl.ds(base_row + b * CHUNK, CHUNK), :])
    rem = ROWS_PER_TILE % CHUNK
    if rem:
        pltpu.sync_copy(
            msg_v.at[pl.ds(0, rem), :],
            slab.at[pl.ds(base_row + (ROWS_PER_TILE // CHUNK) * CHUNK, rem), :],
        )
    plsc.subcore_barrier()

    def _chunk(t, carry):
        base = wid * (CONV_CHUNKS * CHUNK) + t * CHUNK
        pltpu.sync_copy(src_hbm.at[pl.ds(base, CHUNK)], idx_s)
        pltpu.sync_copy(dst_hbm.at[pl.ds(base, CHUNK)], idx_d)
        pltpu.sync_copy(attr_hbm.at[pl.ds(base * 4, CHUNK * 4)],
                        attr_v.at[pl.ds(0, CHUNK * 4)])
        cp = pltpu.async_copy(hw_hbm.at[idx_s], rows_v, sem)

        for g in range(CHUNK // LANES):
            sl = pl.ds(g * LANES, LANES)
            sv = idx_s[sl]
            dv = idx_d[sl]
            norm_v[sl] = plsc.load_gather(rs_v, [sv]) * plsc.load_gather(rs_v, [dv])

        cp.wait()

        @plsc.parallel_loop(0, CHUNK, unroll=4)
        def _edge(j):
            av = attr_v[pl.ds(j * 4, LANES)]
            a0 = av[0]
            a1 = av[1]
            a2 = av[2]
            a3 = av[3]
            nj = norm_v[pl.ds(j, LANES)][0]
            for q in range(EMB // LANES):
                sl = pl.ds(q * LANES, LANES)
                e = (a0 * we_v[0, sl] + a1 * we_v[1, sl]
                     + a2 * we_v[2, sl] + a3 * we_v[3, sl])
                msg_v[j, sl] = jnp.maximum(rows_v[j, sl] + e, 0.0) * nj

        pltpu.sync_copy(msg_v, slab.at[idx_d], add=True)
        return carry

    lax.fori_loop(0, CONV_CHUNKS, _chunk, 0)
    plsc.subcore_barrier()
    pltpu.sync_copy(
        slab.at[pl.ds(sid * ROWS_PER_TILE, ROWS_PER_TILE), :],
        out_hbm.at[cid].at[pl.ds(sid * ROWS_PER_TILE, ROWS_PER_TILE), :],
    )


def _conv(src_pad, dst_pad, attr_pad, rs, hw, we_l):
    k = pl.kernel(
        _conv_body,
        out_type=pltpu.HBM((NCORES, SLAB_ROWS, EMB), jnp.float32),
        mesh=_sc_mesh(),
        compiler_params=pltpu.CompilerParams(needs_layout_passes=False),
        scratch_types=[
            pltpu.VMEM_SHARED((SLAB_ROWS, EMB), jnp.float32),
            pltpu.VMEM((SLAB_ROWS,), jnp.float32),
            pltpu.VMEM((4, EMB), jnp.float32),
            pltpu.VMEM((CHUNK,), jnp.int32),
            pltpu.VMEM((CHUNK,), jnp.int32),
            pltpu.VMEM((CHUNK * 4 + LANES,), jnp.float32),
            pltpu.VMEM((CHUNK + LANES,), jnp.float32),
            pltpu.VMEM((CHUNK, EMB), jnp.float32),
            pltpu.VMEM((CHUNK, EMB), jnp.float32),
            pltpu.SemaphoreType.DMA,
        ],
    )
    return k(src_pad, dst_pad, attr_pad, rs, hw, we_l)


def kernel(x, edge_index, edge_attr, batch, W_enc, b_enc, Wg, bg, We,
           W_pred, b_pred):
    src = edge_index[0]
    dst = edge_index[1]
    iota = jnp.arange(NN, dtype=jnp.int32)

    npad = E_CONV - EE - NN
    src_pad = jnp.concatenate([src, iota, jnp.zeros((npad,), jnp.int32)])
    dst_pad = jnp.concatenate(
        [dst, iota, jnp.full((npad,), DEAD_ROW, jnp.int32)]
    )
    attr_pad = jnp.concatenate(
        [edge_attr, jnp.zeros((NN + npad, 4), jnp.float32)]
    ).reshape(-1)
    dst_deg = jnp.concatenate(
        [dst, jnp.full((E_DEG - EE,), DEAD_ROW, jnp.int32)]
    )

    degp = _deg(dst_deg)
    dcol = degp[:, :, 0].reshape(2, SLAB_ROWS // 128, 128)
    rs = _rs(dcol).reshape(SLAB_ROWS)

    feat = _mm1(x, W_enc, b_enc)

    parts = []
    for i in range(2):
        hw0 = _mm1(feat, Wg[i, 0], bg[i, 0])
        p = _conv(src_pad, dst_pad, attr_pad, rs, hw0, We[i, 0])
        hw1 = _mm2(p[0], p[1], Wg[i, 1], bg[i, 1])
        p2 = _conv(src_pad, dst_pad, attr_pad, rs, hw1, We[i, 1])
        parts.append(p2)

    batch3 = batch.reshape(NN // MBLK, 1, MBLK)
    return _pool(parts[0][0], parts[0][1], parts[1][0], parts[1][1],
                 batch3, W_pred, b_pred)
